# baseline (speedup 1.0000x reference)
import functools

import jax
import jax.numpy as jnp
from jax import lax
from jax.experimental import pallas as pl
from jax.experimental.pallas import tpu as pltpu
from jax.experimental.pallas import tpu_sc as plsc

N_NODES = 10000
N_PAD = 10240
PAD_ROWS = N_PAD - N_NODES
NC = 2
NS = 16
NW = NC * NS
CHUNK = 125
RPT = N_PAD // NS
EPS = 1e-5
ROW_BLK = 1280
NBUF = 8


def _worker(c, s):
    return c * NS + s


def _sc_degree(edges, ones, z, n_chunks):
    mesh = plsc.VectorSubcoreMesh(core_axis_name="c", subcore_axis_name="s")

    @functools.partial(
        pl.kernel,
        out_type=jax.ShapeDtypeStruct((NC, N_PAD, 8), jnp.float32),
        mesh=mesh,
        scratch_types=[
            pltpu.VMEM((n_chunks, CHUNK), jnp.int32),
            pltpu.VMEM((CHUNK, 8), jnp.float32),
            pltpu.VMEM_SHARED((N_PAD, 8), jnp.float32),
            pltpu.SemaphoreType.DMA,
        ],
        compiler_params=pltpu.CompilerParams(use_tc_tiling_on_sc=False),
    )
    def deg_kernel(e_hbm, ones_hbm, z_hbm, out_hbm, idx_v, ones_v, hist_sh,
                   sem):
        c = lax.axis_index("c")
        s = lax.axis_index("s")
        w = _worker(c, s)
        pltpu.sync_copy(e_hbm.at[1, w], idx_v)
        pltpu.sync_copy(ones_hbm, ones_v)
        pltpu.sync_copy(z_hbm, hist_sh.at[pl.ds(s * RPT, RPT)])
        plsc.subcore_barrier()

        def body(j, carry):
            pltpu.async_copy(ones_v, hist_sh.at[idx_v.at[j]], sem, add=True)
            return carry

        lax.fori_loop(0, n_chunks, body, 0)

        def drain(j, carry):
            pltpu.make_async_copy(ones_v, hist_sh.at[idx_v.at[0]], sem).wait()
            return carry

        lax.fori_loop(0, n_chunks, drain, 0)
        plsc.subcore_barrier()
        pltpu.sync_copy(hist_sh.at[pl.ds(s * RPT, RPT)],
                        out_hbm.at[c, pl.ds(s * RPT, RPT)])

    return deg_kernel(edges, ones, z)


def _sc_aggregate(yp, edges, z, n_chunks, d):
    mesh = plsc.VectorSubcoreMesh(core_axis_name="c", subcore_axis_name="s")
    stage = d <= 32

    @functools.partial(
        pl.kernel,
        out_type=jax.ShapeDtypeStruct((NC, N_PAD, d), jnp.float32),
        mesh=mesh,
        scratch_types=[
            pltpu.VMEM((n_chunks, CHUNK), jnp.int32),
            pltpu.VMEM((n_chunks, CHUNK), jnp.int32),
            pltpu.VMEM((NBUF, CHUNK, d), jnp.float32),
            pltpu.VMEM_SHARED((N_PAD, d), jnp.float32),
        ] + ([pltpu.VMEM_SHARED((N_PAD, d), jnp.float32)] if stage else [])
          + [pltpu.SemaphoreType.DMA] * NBUF,
        compiler_params=pltpu.CompilerParams(use_tc_tiling_on_sc=False),
    )
    def agg_kernel(y_hbm, e_hbm, z_hbm, out_hbm,
                   src_v, dst_v, rows_v, acc_sh, *rest):
        c = lax.axis_index("c")
        s = lax.axis_index("s")
        w = _worker(c, s)
        if stage:
            y_sh, *sems = rest
            y_src = y_sh
        else:
            sems = rest
            y_src = y_hbm
        gsems = sems[:NBUF]

        def ygath(j):
            return y_src.at[src_v.at[j]]

        pltpu.sync_copy(e_hbm.at[0, w], src_v)
        pltpu.sync_copy(e_hbm.at[1, w], dst_v)
        if stage:
            pltpu.sync_copy(y_hbm.at[pl.ds(s * RPT, RPT)],
                            y_sh.at[pl.ds(s * RPT, RPT)])
            pltpu.sync_copy(z_hbm, acc_sh.at[pl.ds(s * RPT, RPT)])
            plsc.subcore_barrier()
            for b in range(NBUF):
                pltpu.async_copy(ygath(b), rows_v.at[b], gsems[b])
        else:
            for b in range(NBUF):
                pltpu.async_copy(ygath(b), rows_v.at[b], gsems[b])
            pltpu.sync_copy(z_hbm, acc_sh.at[pl.ds(s * RPT, RPT)])
            plsc.subcore_barrier()

        def gwait(j, b):
            pltpu.make_async_copy(ygath(j), rows_v.at[b], gsems[b]).wait()

        def body(g, carry):
            base = g * NBUF
            for b in range(NBUF):
                j = base + b
                gwait(j, b)
                pltpu.sync_copy(rows_v.at[b], acc_sh.at[dst_v.at[j]],
                                add=True)
                pltpu.async_copy(ygath(j + NBUF), rows_v.at[b], gsems[b])
            return carry

        lax.fori_loop(0, n_chunks // NBUF - 1, body, 0)
        for b in range(NBUF):
            j = n_chunks - NBUF + b
            gwait(j, b)
            pltpu.sync_copy(rows_v.at[b], acc_sh.at[dst_v.at[j]], add=True)
        plsc.subcore_barrier()
        pltpu.sync_copy(acc_sh.at[pl.ds(s * RPT, RPT)],
                        out_hbm.at[c, pl.ds(s * RPT, RPT)])

    return agg_kernel(yp.reshape(N_PAD, d), edges, z)


def _tc_prep(x, w1b):
    def body(x_ref, w_ref, y_ref):
        xm = x_ref[...].reshape(x_ref.shape[0] // 2, 256)
        y_ref[...] = jnp.dot(xm, w_ref[...],
                             preferred_element_type=jnp.float32)

    grid = N_PAD // ROW_BLK
    rb = ROW_BLK // 2
    return pl.pallas_call(
        body,
        grid=(grid,),
        in_specs=[
            pl.BlockSpec((ROW_BLK, 128), lambda i: (i, 0)),
            pl.BlockSpec((256, 128), lambda i: (0, 0)),
        ],
        out_specs=pl.BlockSpec((rb, 128), lambda i: (i, 0)),
        out_shape=jax.ShapeDtypeStruct((N_PAD // 2, 128), jnp.float32),
    )(x, w1b)


def _tc_mid(parts, y1p, dinvp1, dinvp2, b1p, g1p, be1p, w2b4):
    def body(p_ref, y_ref, d1_ref, d2_ref, b_ref, g_ref, be_ref, w_ref,
             o_ref):
        pre = (p_ref[0] + p_ref[1] + y_ref[...]) * d1_ref[...] + b_ref[...]
        bns = g_ref[...] * (1.0 / jnp.sqrt(1.0 + EPS))
        h = jnp.maximum(pre * bns + be_ref[...], 0.0)
        hm = h.reshape(h.shape[0] // 2, 256)
        o_ref[...] = jnp.dot(hm, w_ref[...],
                             preferred_element_type=jnp.float32) * d2_ref[...]

    grid = N_PAD // ROW_BLK
    rb = ROW_BLK // 2
    rq = ROW_BLK // 4
    return pl.pallas_call(
        body,
        grid=(grid,),
        in_specs=[
            pl.BlockSpec((2, rb, 128), lambda i: (0, i, 0)),
            pl.BlockSpec((rb, 128), lambda i: (i, 0)),
            pl.BlockSpec((rb, 128), lambda i: (i, 0)),
            pl.BlockSpec((rq, 128), lambda i: (i, 0)),
            pl.BlockSpec((1, 128), lambda i: (0, 0)),
            pl.BlockSpec((1, 128), lambda i: (0, 0)),
            pl.BlockSpec((1, 128), lambda i: (0, 0)),
            pl.BlockSpec((256, 128), lambda i: (0, 0)),
        ],
        out_specs=pl.BlockSpec((rq, 128), lambda i: (i, 0)),
        out_shape=jax.ShapeDtypeStruct((N_PAD // 4, 128), jnp.float32),
    )(parts, y1p, dinvp1, dinvp2, b1p, g1p, be1p, w2b4)


def _tc_out(parts, y2p, dinvp2, b2p, g2p, be2p, wlq, blq):
    def body(p_ref, y_ref, d_ref, b_ref, g_ref, be_ref, w_ref, bl_ref,
             o_ref):
        pre = (p_ref[0] + p_ref[1] + y_ref[...]) * d_ref[...] + b_ref[...]
        bns = g_ref[...] * (1.0 / jnp.sqrt(1.0 + EPS))
        h = jnp.maximum(pre * bns + be_ref[...], 0.0)
        o_ref[...] = jnp.dot(h, w_ref[...],
                             preferred_element_type=jnp.float32) + bl_ref[...]

    grid = N_PAD // ROW_BLK
    rq = ROW_BLK // 4
    return pl.pallas_call(
        body,
        grid=(grid,),
        in_specs=[
            pl.BlockSpec((2, rq, 128), lambda i: (0, i, 0)),
            pl.BlockSpec((rq, 128), lambda i: (i, 0)),
            pl.BlockSpec((rq, 128), lambda i: (i, 0)),
            pl.BlockSpec((1, 128), lambda i: (0, 0)),
            pl.BlockSpec((1, 128), lambda i: (0, 0)),
            pl.BlockSpec((1, 128), lambda i: (0, 0)),
            pl.BlockSpec((128, 8), lambda i: (0, 0)),
            pl.BlockSpec((1, 8), lambda i: (0, 0)),
        ],
        out_specs=pl.BlockSpec((rq, 8), lambda i: (i, 0)),
        out_shape=jax.ShapeDtypeStruct((N_PAD // 4, 8), jnp.float32),
    )(parts, y2p, dinvp2, b2p, g2p, be2p, wlq, blq)


def kernel(x, edge_index, W1, b1, g1, be1, W2, b2, g2, be2, Wl, bl):
    src = edge_index[0].astype(jnp.int32)
    dst = edge_index[1].astype(jnp.int32)
    e = src.shape[0]
    n_chunks = -(-e // (NW * CHUNK))
    n_chunks = -(-n_chunks // NBUF) * NBUF
    epad = NW * n_chunks * CHUNK
    pad_n = epad - e
    if pad_n:
        pad_idx = N_NODES + (jnp.arange(pad_n, dtype=jnp.int32) % PAD_ROWS)
        src = jnp.concatenate([src, pad_idx])
        dst = jnp.concatenate([dst, pad_idx])
        edges = jnp.stack([src, dst]).reshape(2, NW, n_chunks, CHUNK)
    else:
        edges = jnp.stack([src, dst]).reshape(2, NW, n_chunks, CHUNK)

    xp = jnp.zeros((N_PAD, 128), jnp.float32).at[:N_NODES].set(x)
    ones8 = jnp.ones((CHUNK, 8), jnp.float32)
    z8 = jnp.zeros((RPT, 8), jnp.float32)
    z64 = jnp.zeros((RPT, 64), jnp.float32)
    z32 = jnp.zeros((RPT, 32), jnp.float32)

    w1b = jnp.zeros((256, 128), jnp.float32)
    w1b = w1b.at[:128, :64].set(W1).at[128:, 64:].set(W1)
    w2b4 = jnp.zeros((256, 128), jnp.float32)
    for i in range(4):
        w2b4 = w2b4.at[i * 64:(i + 1) * 64, i * 32:(i + 1) * 32].set(W2)
    wlq = jnp.zeros((128, 8), jnp.float32)
    for i in range(4):
        wlq = wlq.at[i * 32:(i + 1) * 32, i * 2:(i + 1) * 2].set(Wl)
    b1p = jnp.tile(b1, 2).reshape(1, 128)
    g1p = jnp.tile(g1, 2).reshape(1, 128)
    be1p = jnp.tile(be1, 2).reshape(1, 128)
    b2p = jnp.tile(b2, 4).reshape(1, 128)
    g2p = jnp.tile(g2, 4).reshape(1, 128)
    be2p = jnp.tile(be2, 4).reshape(1, 128)
    blq = jnp.tile(bl, 4).reshape(1, 8)

    degp = _sc_degree(edges, ones8, z8, n_chunks)
    xwp = _tc_prep(xp, w1b)
    deg = degp[0, :, 0] + degp[1, :, 0] + 1.0
    dinv = lax.rsqrt(deg)
    dinvp1 = jnp.repeat(dinv, 64).reshape(N_PAD // 2, 128)
    dinvp2 = jnp.repeat(dinv, 32).reshape(N_PAD // 4, 128)

    y1p = xwp * dinvp1
    parts1 = _sc_aggregate(y1p, edges, z64, n_chunks, 64)
    parts1 = parts1.reshape(NC, N_PAD // 2, 128)
    y2p = _tc_mid(parts1, y1p, dinvp1, dinvp2, b1p, g1p, be1p, w2b4)
    parts2 = _sc_aggregate(y2p, edges, z32, n_chunks, 32)
    parts2 = parts2.reshape(NC, N_PAD // 4, 128)
    out = _tc_out(parts2, y2p, dinvp2, b2p, g2p, be2p, wlq, blq)
    return out.reshape(N_PAD, 2)[:N_NODES]


# speedup vs baseline: 1.1008x; 1.1008x over previous
import functools

import jax
import jax.numpy as jnp
from jax import lax
from jax.experimental import pallas as pl
from jax.experimental.pallas import tpu as pltpu
from jax.experimental.pallas import tpu_sc as plsc

N_NODES = 10000
N_PAD = 10240
PAD_ROWS = N_PAD - N_NODES
NC = 2
NS = 16
NW = NC * NS
CHUNK = 125
RPT = N_PAD // NS
EPS = 1e-5
ROW_BLK = 2560
NBUF = 8


def _worker(c, s):
    return c * NS + s


def _sc_degree(edges, ones, z, n_chunks):
    mesh = plsc.VectorSubcoreMesh(core_axis_name="c", subcore_axis_name="s")

    @functools.partial(
        pl.kernel,
        out_type=jax.ShapeDtypeStruct((NC, N_PAD, 8), jnp.float32),
        mesh=mesh,
        scratch_types=[
            pltpu.VMEM((n_chunks, CHUNK), jnp.int32),
            pltpu.VMEM((CHUNK, 8), jnp.float32),
            pltpu.VMEM_SHARED((N_PAD, 8), jnp.float32),
            pltpu.SemaphoreType.DMA,
        ],
        compiler_params=pltpu.CompilerParams(use_tc_tiling_on_sc=False),
    )
    def deg_kernel(e_hbm, ones_hbm, z_hbm, out_hbm, idx_v, ones_v, hist_sh,
                   sem):
        c = lax.axis_index("c")
        s = lax.axis_index("s")
        w = _worker(c, s)
        pltpu.sync_copy(e_hbm.at[1, w], idx_v)
        pltpu.sync_copy(ones_hbm, ones_v)
        pltpu.sync_copy(z_hbm, hist_sh.at[pl.ds(s * RPT, RPT)])
        plsc.subcore_barrier()

        def body(j, carry):
            pltpu.async_copy(ones_v, hist_sh.at[idx_v.at[j]], sem, add=True)
            return carry

        lax.fori_loop(0, n_chunks, body, 0)

        def drain(j, carry):
            pltpu.make_async_copy(ones_v, hist_sh.at[idx_v.at[0]], sem).wait()
            return carry

        lax.fori_loop(0, n_chunks, drain, 0)
        plsc.subcore_barrier()
        pltpu.sync_copy(hist_sh.at[pl.ds(s * RPT, RPT)],
                        out_hbm.at[c, pl.ds(s * RPT, RPT)])

    return deg_kernel(edges, ones, z)


def _sc_aggregate(yp, edges, z, n_chunks, d):
    mesh = plsc.VectorSubcoreMesh(core_axis_name="c", subcore_axis_name="s")
    nbuf = NBUF if d > 32 else 2 * NBUF

    @functools.partial(
        pl.kernel,
        out_type=jax.ShapeDtypeStruct((NC, N_PAD, d), jnp.float32),
        mesh=mesh,
        scratch_types=[
            pltpu.VMEM((n_chunks, CHUNK), jnp.int32),
            pltpu.VMEM((n_chunks, CHUNK), jnp.int32),
            pltpu.VMEM((nbuf, CHUNK, d), jnp.float32),
            pltpu.VMEM_SHARED((N_PAD, d), jnp.float32),
        ] + [pltpu.SemaphoreType.DMA] * nbuf,
        compiler_params=pltpu.CompilerParams(use_tc_tiling_on_sc=False),
    )
    def agg_kernel(y_hbm, e_hbm, z_hbm, out_hbm,
                   src_v, dst_v, rows_v, acc_sh, *sems):
        c = lax.axis_index("c")
        s = lax.axis_index("s")
        w = _worker(c, s)
        gsems = sems[:nbuf]

        def ygath(j):
            return y_hbm.at[src_v.at[j]]

        pltpu.sync_copy(e_hbm.at[0, w], src_v)
        pltpu.sync_copy(e_hbm.at[1, w], dst_v)
        for b in range(nbuf):
            pltpu.async_copy(ygath(b), rows_v.at[b], gsems[b])
        pltpu.sync_copy(z_hbm, acc_sh.at[pl.ds(s * RPT, RPT)])
        plsc.subcore_barrier()

        def gwait(j, b):
            pltpu.make_async_copy(ygath(j), rows_v.at[b], gsems[b]).wait()

        def body(g, carry):
            base = g * nbuf
            for b in range(nbuf):
                j = base + b
                gwait(j, b)
                pltpu.sync_copy(rows_v.at[b], acc_sh.at[dst_v.at[j]],
                                add=True)
                pltpu.async_copy(ygath(j + nbuf), rows_v.at[b], gsems[b])
            return carry

        lax.fori_loop(0, n_chunks // nbuf - 1, body, 0)
        for b in range(nbuf):
            j = n_chunks - nbuf + b
            gwait(j, b)
            pltpu.sync_copy(rows_v.at[b], acc_sh.at[dst_v.at[j]], add=True)
        plsc.subcore_barrier()
        pltpu.sync_copy(acc_sh.at[pl.ds(s * RPT, RPT)],
                        out_hbm.at[c, pl.ds(s * RPT, RPT)])

    return agg_kernel(yp.reshape(N_PAD, d), edges, z)


def _tc_prep(x, w1b):
    def body(x_ref, w_ref, y_ref):
        xm = x_ref[...].reshape(x_ref.shape[0] // 2, 256)
        y_ref[...] = jnp.dot(xm, w_ref[...],
                             preferred_element_type=jnp.float32)

    grid = N_PAD // ROW_BLK
    rb = ROW_BLK // 2
    return pl.pallas_call(
        body,
        grid=(grid,),
        in_specs=[
            pl.BlockSpec((ROW_BLK, 128), lambda i: (i, 0)),
            pl.BlockSpec((256, 128), lambda i: (0, 0)),
        ],
        out_specs=pl.BlockSpec((rb, 128), lambda i: (i, 0)),
        out_shape=jax.ShapeDtypeStruct((N_PAD // 2, 128), jnp.float32),
    )(x, w1b)


def _tc_mid(parts, y1p, dinvp1, dinvp2, b1p, g1p, be1p, w2b4):
    def body(p_ref, y_ref, d1_ref, d2_ref, b_ref, g_ref, be_ref, w_ref,
             o_ref):
        pre = (p_ref[0] + p_ref[1] + y_ref[...]) * d1_ref[...] + b_ref[...]
        bns = g_ref[...] * (1.0 / jnp.sqrt(1.0 + EPS))
        h = jnp.maximum(pre * bns + be_ref[...], 0.0)
        hm = h.reshape(h.shape[0] // 2, 256)
        o_ref[...] = jnp.dot(hm, w_ref[...],
                             preferred_element_type=jnp.float32) * d2_ref[...]

    grid = N_PAD // ROW_BLK
    rb = ROW_BLK // 2
    rq = ROW_BLK // 4
    return pl.pallas_call(
        body,
        grid=(grid,),
        in_specs=[
            pl.BlockSpec((2, rb, 128), lambda i: (0, i, 0)),
            pl.BlockSpec((rb, 128), lambda i: (i, 0)),
            pl.BlockSpec((rb, 128), lambda i: (i, 0)),
            pl.BlockSpec((rq, 128), lambda i: (i, 0)),
            pl.BlockSpec((1, 128), lambda i: (0, 0)),
            pl.BlockSpec((1, 128), lambda i: (0, 0)),
            pl.BlockSpec((1, 128), lambda i: (0, 0)),
            pl.BlockSpec((256, 128), lambda i: (0, 0)),
        ],
        out_specs=pl.BlockSpec((rq, 128), lambda i: (i, 0)),
        out_shape=jax.ShapeDtypeStruct((N_PAD // 4, 128), jnp.float32),
    )(parts, y1p, dinvp1, dinvp2, b1p, g1p, be1p, w2b4)


def _tc_out(parts, y2p, dinvp2, b2p, g2p, be2p, wlq, blq):
    def body(p_ref, y_ref, d_ref, b_ref, g_ref, be_ref, w_ref, bl_ref,
             o_ref):
        pre = (p_ref[0] + p_ref[1] + y_ref[...]) * d_ref[...] + b_ref[...]
        bns = g_ref[...] * (1.0 / jnp.sqrt(1.0 + EPS))
        h = jnp.maximum(pre * bns + be_ref[...], 0.0)
        o_ref[...] = jnp.dot(h, w_ref[...],
                             preferred_element_type=jnp.float32) + bl_ref[...]

    grid = N_PAD // ROW_BLK
    rq = ROW_BLK // 4
    return pl.pallas_call(
        body,
        grid=(grid,),
        in_specs=[
            pl.BlockSpec((2, rq, 128), lambda i: (0, i, 0)),
            pl.BlockSpec((rq, 128), lambda i: (i, 0)),
            pl.BlockSpec((rq, 128), lambda i: (i, 0)),
            pl.BlockSpec((1, 128), lambda i: (0, 0)),
            pl.BlockSpec((1, 128), lambda i: (0, 0)),
            pl.BlockSpec((1, 128), lambda i: (0, 0)),
            pl.BlockSpec((128, 8), lambda i: (0, 0)),
            pl.BlockSpec((1, 8), lambda i: (0, 0)),
        ],
        out_specs=pl.BlockSpec((rq, 8), lambda i: (i, 0)),
        out_shape=jax.ShapeDtypeStruct((N_PAD // 4, 8), jnp.float32),
    )(parts, y2p, dinvp2, b2p, g2p, be2p, wlq, blq)


def kernel(x, edge_index, W1, b1, g1, be1, W2, b2, g2, be2, Wl, bl):
    src = edge_index[0].astype(jnp.int32)
    dst = edge_index[1].astype(jnp.int32)
    e = src.shape[0]
    n_chunks = -(-e // (NW * CHUNK))
    n_chunks = -(-n_chunks // (2 * NBUF)) * (2 * NBUF)
    epad = NW * n_chunks * CHUNK
    pad_n = epad - e
    if pad_n:
        pad_idx = N_NODES + (jnp.arange(pad_n, dtype=jnp.int32) % PAD_ROWS)
        src = jnp.concatenate([src, pad_idx])
        dst = jnp.concatenate([dst, pad_idx])
        edges = jnp.stack([src, dst]).reshape(2, NW, n_chunks, CHUNK)
    else:
        edges = jnp.stack([src, dst]).reshape(2, NW, n_chunks, CHUNK)

    xp = jnp.zeros((N_PAD, 128), jnp.float32).at[:N_NODES].set(x)
    ones8 = jnp.ones((CHUNK, 8), jnp.float32)
    z8 = jnp.zeros((RPT, 8), jnp.float32)
    z64 = jnp.zeros((RPT, 64), jnp.float32)
    z32 = jnp.zeros((RPT, 32), jnp.float32)

    w1b = jnp.zeros((256, 128), jnp.float32)
    w1b = w1b.at[:128, :64].set(W1).at[128:, 64:].set(W1)
    w2b4 = jnp.zeros((256, 128), jnp.float32)
    for i in range(4):
        w2b4 = w2b4.at[i * 64:(i + 1) * 64, i * 32:(i + 1) * 32].set(W2)
    wlq = jnp.zeros((128, 8), jnp.float32)
    for i in range(4):
        wlq = wlq.at[i * 32:(i + 1) * 32, i * 2:(i + 1) * 2].set(Wl)
    b1p = jnp.tile(b1, 2).reshape(1, 128)
    g1p = jnp.tile(g1, 2).reshape(1, 128)
    be1p = jnp.tile(be1, 2).reshape(1, 128)
    b2p = jnp.tile(b2, 4).reshape(1, 128)
    g2p = jnp.tile(g2, 4).reshape(1, 128)
    be2p = jnp.tile(be2, 4).reshape(1, 128)
    blq = jnp.tile(bl, 4).reshape(1, 8)

    degp = _sc_degree(edges, ones8, z8, n_chunks)
    xwp = _tc_prep(xp, w1b)
    deg = degp[0, :, 0] + degp[1, :, 0] + 1.0
    dinv = lax.rsqrt(deg)
    dinvp1 = jnp.repeat(dinv, 64).reshape(N_PAD // 2, 128)
    dinvp2 = jnp.repeat(dinv, 32).reshape(N_PAD // 4, 128)

    y1p = xwp * dinvp1
    parts1 = _sc_aggregate(y1p, edges, z64, n_chunks, 64)
    parts1 = parts1.reshape(NC, N_PAD // 2, 128)
    y2p = _tc_mid(parts1, y1p, dinvp1, dinvp2, b1p, g1p, be1p, w2b4)
    parts2 = _sc_aggregate(y2p, edges, z32, n_chunks, 32)
    parts2 = parts2.reshape(NC, N_PAD // 4, 128)
    out = _tc_out(parts2, y2p, dinvp2, b2p, g2p, be2p, wlq, blq)
    return out.reshape(N_PAD, 2)[:N_NODES]
